# baseline (speedup 1.0000x reference)
import jax
import jax.numpy as jnp
from jax import lax
from jax.experimental import pallas as pl
from jax.experimental.pallas import tpu as pltpu
from jax.experimental.pallas import tpu_sc as plsc

_NH = 12
_NN = 20736
_NU = 3312
_FLAT = _NH * _NN
_NC = 2
_CHUNK = _NN // 2
_NWORK = 2 * _NH


def _sc_gather_body(table_hbm, idx_hbm, out_hbm, table_v, idx_v, out_v):
    wid = lax.axis_index("s") * _NC + lax.axis_index("c")

    @pl.when(wid < _NWORK)
    def _():
        h = wid // 2
        c0 = (wid % 2) * _CHUNK
        pltpu.sync_copy(table_hbm, table_v)
        pltpu.sync_copy(idx_hbm.at[pl.ds(c0, _CHUNK)], idx_v)

        def body(j, carry):
            ivec = idx_v[pl.ds(j * 16, 16)] * _NH + h
            out_v[pl.ds(j * 16, 16)] = plsc.load_gather(table_v, [ivec])
            return carry

        lax.fori_loop(0, _CHUNK // 16, body, 0)
        pltpu.sync_copy(out_v, out_hbm.at[pl.ds(h * _NN + c0, _CHUNK)])


_sc_gather = pl.kernel(
    _sc_gather_body,
    out_type=jax.ShapeDtypeStruct((_FLAT,), jnp.float32),
    mesh=plsc.VectorSubcoreMesh(core_axis_name="c", subcore_axis_name="s"),
    compiler_params=pltpu.CompilerParams(needs_layout_passes=False),
    scratch_types=[
        pltpu.VMEM((_NU * _NH,), jnp.float32),
        pltpu.VMEM((_CHUNK,), jnp.int32),
        pltpu.VMEM((_CHUNK,), jnp.float32),
    ],
)


_ROWS = 2
_NCH = 240 // _ROWS
_K = 8


def _add_manual_body(attn_hbm, bias_v, out_hbm, ibuf, obuf, in_sem, out_sem):
    aflat = attn_hbm.reshape(240, _FLAT)
    oflat = out_hbm.reshape(240, _FLAT)

    def in_copy(c, slot):
        return pltpu.make_async_copy(
            aflat.at[pl.ds(c * _ROWS, _ROWS)], ibuf.at[slot],
            in_sem.at[slot])

    def out_copy(c, slot):
        return pltpu.make_async_copy(
            obuf.at[slot], oflat.at[pl.ds(c * _ROWS, _ROWS)],
            out_sem.at[slot])

    for k in range(_K):
        in_copy(k, k).start()

    def body(j, carry):
        slot = lax.rem(j, _K)
        in_copy(j, slot).wait()

        @pl.when(j >= _K)
        def _():
            out_copy(j - _K, slot).wait()

        obuf[slot] = ibuf[slot] + bias_v[...]
        out_copy(j, slot).start()

        @pl.when(j + _K < _NCH)
        def _():
            in_copy(j + _K, slot).start()

        return carry

    lax.fori_loop(0, _NCH, body, 0)
    for k in range(_K):
        c = _NCH - _K + k
        out_copy(c, c % _K).wait()


def _add_body4(attn_ref, bias_ref, out_ref):
    out_ref[...] = attn_ref[...] + bias_ref[...]


def _tc_add4(attn, bias3, blk=4):
    b = attn.shape[0]
    return pl.pallas_call(
        _add_body4,
        grid=(b // blk,),
        in_specs=[
            pl.BlockSpec((blk, _NH, 144, 144), lambda i: (i, 0, 0, 0)),
            pl.BlockSpec((_NH, 144, 144), lambda i: (0, 0, 0)),
        ],
        out_specs=pl.BlockSpec((blk, _NH, 144, 144), lambda i: (i, 0, 0, 0)),
        out_shape=jax.ShapeDtypeStruct(attn.shape, jnp.float32),
        compiler_params=pltpu.CompilerParams(
            dimension_semantics=("arbitrary",)),
    )(attn, bias3)


def kernel(attn, earth_specific_bias, position_index, window_type_index):
    w = jnp.asarray(window_type_index, jnp.int32)
    table = lax.dynamic_index_in_dim(
        earth_specific_bias, w, axis=1, keepdims=False)
    a2 = attn.reshape(2880, _NN)

    def _copy_body(a_ref, o_ref):
        o_ref[...] = a_ref[...]
    blk = 24
    out = pl.pallas_call(
        _copy_body,
        grid=(2880 // blk,),
        in_specs=[pl.BlockSpec((blk, _NN), lambda i: (i, 0))],
        out_specs=pl.BlockSpec((blk, _NN), lambda i: (i, 0)),
        out_shape=jax.ShapeDtypeStruct((2880, _NN), jnp.float32),
    )(a2)
    return out.reshape(attn.shape)


# speedup vs baseline: 10.9502x; 10.9502x over previous
import jax
import jax.numpy as jnp
from jax import lax
from jax.experimental import pallas as pl
from jax.experimental.pallas import tpu as pltpu
from jax.experimental.pallas import tpu_sc as plsc

_NH = 12
_NN = 20736
_NU = 3312
_FLAT = _NH * _NN
_NC = 2
_CHUNK = _NN // 2
_NWORK = 2 * _NH


def _sc_gather_body(table_hbm, idx_hbm, out_hbm, table_v, idx_v, out_v):
    wid = lax.axis_index("s") * _NC + lax.axis_index("c")

    @pl.when(wid < _NWORK)
    def _():
        h = wid // 2
        c0 = (wid % 2) * _CHUNK
        pltpu.sync_copy(table_hbm, table_v)
        pltpu.sync_copy(idx_hbm.at[pl.ds(c0, _CHUNK)], idx_v)

        def body(j, carry):
            ivec = idx_v[pl.ds(j * 16, 16)] * _NH + h
            out_v[pl.ds(j * 16, 16)] = plsc.load_gather(table_v, [ivec])
            return carry

        lax.fori_loop(0, _CHUNK // 16, body, 0)
        pltpu.sync_copy(out_v, out_hbm.at[pl.ds(h * _NN + c0, _CHUNK)])


_sc_gather = pl.kernel(
    _sc_gather_body,
    out_type=jax.ShapeDtypeStruct((_FLAT,), jnp.float32),
    mesh=plsc.VectorSubcoreMesh(core_axis_name="c", subcore_axis_name="s"),
    compiler_params=pltpu.CompilerParams(needs_layout_passes=False),
    scratch_types=[
        pltpu.VMEM((_NU * _NH,), jnp.float32),
        pltpu.VMEM((_CHUNK,), jnp.int32),
        pltpu.VMEM((_CHUNK,), jnp.float32),
    ],
)


_ROWS = 2
_NCH = 240 // _ROWS
_K = 8


def _add_manual_body(attn_hbm, bias_v, out_hbm, ibuf, obuf, in_sem, out_sem):
    aflat = attn_hbm.reshape(240, _FLAT)
    oflat = out_hbm.reshape(240, _FLAT)

    def in_copy(c, slot):
        return pltpu.make_async_copy(
            aflat.at[pl.ds(c * _ROWS, _ROWS)], ibuf.at[slot],
            in_sem.at[slot])

    def out_copy(c, slot):
        return pltpu.make_async_copy(
            obuf.at[slot], oflat.at[pl.ds(c * _ROWS, _ROWS)],
            out_sem.at[slot])

    for k in range(_K):
        in_copy(k, k).start()

    def body(j, carry):
        slot = lax.rem(j, _K)
        in_copy(j, slot).wait()

        @pl.when(j >= _K)
        def _():
            out_copy(j - _K, slot).wait()

        obuf[slot] = ibuf[slot] + bias_v[...]
        out_copy(j, slot).start()

        @pl.when(j + _K < _NCH)
        def _():
            in_copy(j + _K, slot).start()

        return carry

    lax.fori_loop(0, _NCH, body, 0)
    for k in range(_K):
        c = _NCH - _K + k
        out_copy(c, c % _K).wait()


def _add_body_t(attn_ref, bias_ref, out_ref):
    out_ref[...] = attn_ref[...] + bias_ref[...][..., None]


def _tc_add_t(attn_t, bias3, b, ich=16):
    return pl.pallas_call(
        _add_body_t,
        grid=(_NH, 144 // ich),
        in_specs=[
            pl.BlockSpec((1, ich, 144, b), lambda h, i: (h, i, 0, 0)),
            pl.BlockSpec((1, ich, 144), lambda h, i: (h, i, 0)),
        ],
        out_specs=pl.BlockSpec((1, ich, 144, b), lambda h, i: (h, i, 0, 0)),
        out_shape=jax.ShapeDtypeStruct((_NH, 144, 144, b), jnp.float32),
        compiler_params=pltpu.CompilerParams(
            dimension_semantics=("arbitrary", "arbitrary")),
    )(attn_t, bias3)


def kernel(attn, earth_specific_bias, position_index, window_type_index):
    w = jnp.asarray(window_type_index, jnp.int32)
    table = lax.dynamic_index_in_dim(
        earth_specific_bias, w, axis=1, keepdims=False)
    bias_flat = _sc_gather(table.reshape(-1), position_index.astype(jnp.int32))
    b = attn.shape[0]
    attn_t = jnp.transpose(attn, (1, 2, 3, 0))
    out_t = _tc_add_t(attn_t, bias_flat.reshape(_NH, 144, 144), b)
    return jnp.transpose(out_t, (3, 0, 1, 2))


# speedup vs baseline: 12.2707x; 1.1206x over previous
import jax
import jax.numpy as jnp
from jax import lax
from jax.experimental import pallas as pl
from jax.experimental.pallas import tpu as pltpu
from jax.experimental.pallas import tpu_sc as plsc

_NH = 12
_NN = 20736
_NU = 3312
_FLAT = _NH * _NN
_NC = 2
_CHUNK = _NN // 2
_NWORK = 2 * _NH


def _sc_gather_body(table_hbm, idx_hbm, out_hbm, table_v, idx_v, out_v):
    wid = lax.axis_index("s") * _NC + lax.axis_index("c")

    @pl.when(wid < _NWORK)
    def _():
        h = wid // 2
        c0 = (wid % 2) * _CHUNK
        pltpu.sync_copy(table_hbm, table_v)
        pltpu.sync_copy(idx_hbm.at[pl.ds(c0, _CHUNK)], idx_v)

        base = h * _NU

        def body(j, carry):
            ivec = idx_v[pl.ds(j * 16, 16)] + base
            out_v[pl.ds(j * 16, 16)] = plsc.load_gather(table_v, [ivec])
            return carry

        lax.fori_loop(0, _CHUNK // 16, body, 0)
        pltpu.sync_copy(out_v, out_hbm.at[pl.ds(h * _NN + c0, _CHUNK)])


_sc_gather = pl.kernel(
    _sc_gather_body,
    out_type=jax.ShapeDtypeStruct((_FLAT,), jnp.float32),
    mesh=plsc.VectorSubcoreMesh(core_axis_name="c", subcore_axis_name="s"),
    compiler_params=pltpu.CompilerParams(needs_layout_passes=False),
    scratch_types=[
        pltpu.VMEM((_NU * _NH,), jnp.float32),
        pltpu.VMEM((_CHUNK,), jnp.int32),
        pltpu.VMEM((_CHUNK,), jnp.float32),
    ],
)


_ROWS = 2
_NCH = 240 // _ROWS
_K = 8


def _add_manual_body(attn_hbm, bias_v, out_hbm, ibuf, obuf, in_sem, out_sem):
    aflat = attn_hbm.reshape(240, _FLAT)
    oflat = out_hbm.reshape(240, _FLAT)

    def in_copy(c, slot):
        return pltpu.make_async_copy(
            aflat.at[pl.ds(c * _ROWS, _ROWS)], ibuf.at[slot],
            in_sem.at[slot])

    def out_copy(c, slot):
        return pltpu.make_async_copy(
            obuf.at[slot], oflat.at[pl.ds(c * _ROWS, _ROWS)],
            out_sem.at[slot])

    for k in range(_K):
        in_copy(k, k).start()

    def body(j, carry):
        slot = lax.rem(j, _K)
        in_copy(j, slot).wait()

        @pl.when(j >= _K)
        def _():
            out_copy(j - _K, slot).wait()

        obuf[slot] = ibuf[slot] + bias_v[...]
        out_copy(j, slot).start()

        @pl.when(j + _K < _NCH)
        def _():
            in_copy(j + _K, slot).start()

        return carry

    lax.fori_loop(0, _NCH, body, 0)
    for k in range(_K):
        c = _NCH - _K + k
        out_copy(c, c % _K).wait()


def _add_body_t(attn_ref, bias_ref, out_ref):
    out_ref[...] = attn_ref[...] + bias_ref[...][..., None]


def _tc_add_t(attn_t, bias3, b, ich=16):
    return pl.pallas_call(
        _add_body_t,
        grid=(_NH, 144 // ich),
        in_specs=[
            pl.BlockSpec((1, ich, 144, b), lambda h, i: (h, i, 0, 0)),
            pl.BlockSpec((1, ich, 144), lambda h, i: (h, i, 0)),
        ],
        out_specs=pl.BlockSpec((1, ich, 144, b), lambda h, i: (h, i, 0, 0)),
        out_shape=jax.ShapeDtypeStruct((_NH, 144, 144, b), jnp.float32),
        compiler_params=pltpu.CompilerParams(
            dimension_semantics=("arbitrary", "arbitrary")),
    )(attn_t, bias3)


def kernel(attn, earth_specific_bias, position_index, window_type_index):
    w = jnp.asarray(window_type_index, jnp.int32)
    esb_t = jnp.transpose(earth_specific_bias, (2, 0, 1))
    table = lax.dynamic_index_in_dim(esb_t, w, axis=2, keepdims=False)
    bias_flat = _sc_gather(table.reshape(-1), position_index.astype(jnp.int32))
    b = attn.shape[0]
    attn_t = jnp.transpose(attn, (1, 2, 3, 0))
    out_t = _tc_add_t(attn_t, bias_flat.reshape(_NH, 144, 144), b)
    return jnp.transpose(out_t, (3, 0, 1, 2))


# speedup vs baseline: 13.1728x; 1.0735x over previous
import jax
import jax.numpy as jnp
from jax import lax
from jax.experimental import pallas as pl
from jax.experimental.pallas import tpu as pltpu
from jax.experimental.pallas import tpu_sc as plsc

_NH = 12
_NN = 20736
_NU = 3312
_FLAT = _NH * _NN
_NC = 2
_CHUNK = _NN // 2
_NWORK = 2 * _NH


def _sc_gather_body(table_hbm, idx_hbm, out_hbm, table_v, idx_v, out_v):
    wid = lax.axis_index("s") * _NC + lax.axis_index("c")

    @pl.when(wid < _NWORK)
    def _():
        h = wid // 2
        c0 = (wid % 2) * _CHUNK
        pltpu.sync_copy(table_hbm, table_v)
        pltpu.sync_copy(idx_hbm.at[pl.ds(c0, _CHUNK)], idx_v)

        base = h * _NU

        def body(j, carry):
            ivec = idx_v[pl.ds(j * 16, 16)] + base
            out_v[pl.ds(j * 16, 16)] = plsc.load_gather(table_v, [ivec])
            return carry

        lax.fori_loop(0, _CHUNK // 16, body, 0)
        pltpu.sync_copy(out_v, out_hbm.at[pl.ds(h * _NN + c0, _CHUNK)])


_sc_gather = pl.kernel(
    _sc_gather_body,
    out_type=jax.ShapeDtypeStruct((_FLAT,), jnp.float32),
    mesh=plsc.VectorSubcoreMesh(core_axis_name="c", subcore_axis_name="s"),
    compiler_params=pltpu.CompilerParams(needs_layout_passes=False),
    scratch_types=[
        pltpu.VMEM((_NU * _NH,), jnp.float32),
        pltpu.VMEM((_CHUNK,), jnp.int32),
        pltpu.VMEM((_CHUNK,), jnp.float32),
    ],
)


_ROWS = 2
_NCH = 240 // _ROWS
_K = 8


def _add_manual_body(attn_hbm, bias_v, out_hbm, ibuf, obuf, in_sem, out_sem):
    aflat = attn_hbm.reshape(240, _FLAT)
    oflat = out_hbm.reshape(240, _FLAT)

    def in_copy(c, slot):
        return pltpu.make_async_copy(
            aflat.at[pl.ds(c * _ROWS, _ROWS)], ibuf.at[slot],
            in_sem.at[slot])

    def out_copy(c, slot):
        return pltpu.make_async_copy(
            obuf.at[slot], oflat.at[pl.ds(c * _ROWS, _ROWS)],
            out_sem.at[slot])

    for k in range(_K):
        in_copy(k, k).start()

    def body(j, carry):
        slot = lax.rem(j, _K)
        in_copy(j, slot).wait()

        @pl.when(j >= _K)
        def _():
            out_copy(j - _K, slot).wait()

        obuf[slot] = ibuf[slot] + bias_v[...]
        out_copy(j, slot).start()

        @pl.when(j + _K < _NCH)
        def _():
            in_copy(j + _K, slot).start()

        return carry

    lax.fori_loop(0, _NCH, body, 0)
    for k in range(_K):
        c = _NCH - _K + k
        out_copy(c, c % _K).wait()


def _add_body_t(attn_ref, bias_ref, out_ref):
    out_ref[...] = attn_ref[...] + bias_ref[...][..., None]


def _tc_add_t(attn_t, bias3, b, ich=48):
    return pl.pallas_call(
        _add_body_t,
        grid=(_NH, 144 // ich),
        in_specs=[
            pl.BlockSpec((1, ich, 144, b), lambda h, i: (h, i, 0, 0)),
            pl.BlockSpec((1, ich, 144), lambda h, i: (h, i, 0)),
        ],
        out_specs=pl.BlockSpec((1, ich, 144, b), lambda h, i: (h, i, 0, 0)),
        out_shape=jax.ShapeDtypeStruct((_NH, 144, 144, b), jnp.float32),
        compiler_params=pltpu.CompilerParams(
            dimension_semantics=("arbitrary", "arbitrary")),
    )(attn_t, bias3)


def kernel(attn, earth_specific_bias, position_index, window_type_index):
    w = jnp.asarray(window_type_index, jnp.int32)
    esb_t = jnp.transpose(earth_specific_bias, (2, 0, 1))
    table = lax.dynamic_index_in_dim(esb_t, w, axis=2, keepdims=False)
    bias_flat = _sc_gather(table.reshape(-1), position_index.astype(jnp.int32))
    b = attn.shape[0]
    attn_t = jnp.transpose(attn, (1, 2, 3, 0))
    out_t = _tc_add_t(attn_t, bias_flat.reshape(_NH, 144, 144), b)
    return jnp.transpose(out_t, (3, 0, 1, 2))


# speedup vs baseline: 13.2385x; 1.0050x over previous
import jax
import jax.numpy as jnp
from jax import lax
from jax.experimental import pallas as pl
from jax.experimental.pallas import tpu as pltpu
from jax.experimental.pallas import tpu_sc as plsc

_NH = 12
_NN = 20736
_NU = 3312
_FLAT = _NH * _NN
_NC = 2
_CHUNK = _NN // 2
_NWORK = 2 * _NH


def _sc_gather_body(table_hbm, idx_hbm, out_hbm, table_v, idx_v, out_v):
    wid = lax.axis_index("s") * _NC + lax.axis_index("c")

    @pl.when(wid < _NWORK)
    def _():
        h = wid // 2
        c0 = (wid % 2) * _CHUNK
        pltpu.sync_copy(table_hbm, table_v)
        pltpu.sync_copy(idx_hbm.at[pl.ds(c0, _CHUNK)], idx_v)

        base = h * _NU

        def body(j, carry):
            ivec = idx_v[pl.ds(j * 16, 16)] + base
            out_v[pl.ds(j * 16, 16)] = plsc.load_gather(table_v, [ivec])
            return carry

        lax.fori_loop(0, _CHUNK // 16, body, 0)
        pltpu.sync_copy(out_v, out_hbm.at[pl.ds(h * _NN + c0, _CHUNK)])


_sc_gather = pl.kernel(
    _sc_gather_body,
    out_type=jax.ShapeDtypeStruct((_FLAT,), jnp.float32),
    mesh=plsc.VectorSubcoreMesh(core_axis_name="c", subcore_axis_name="s"),
    compiler_params=pltpu.CompilerParams(needs_layout_passes=False),
    scratch_types=[
        pltpu.VMEM((_NU * _NH,), jnp.float32),
        pltpu.VMEM((_CHUNK,), jnp.int32),
        pltpu.VMEM((_CHUNK,), jnp.float32),
    ],
)


_ROWS = 2
_NCH = 240 // _ROWS
_K = 8


def _add_manual_body(attn_hbm, bias_v, out_hbm, ibuf, obuf, in_sem, out_sem):
    aflat = attn_hbm.reshape(240, _FLAT)
    oflat = out_hbm.reshape(240, _FLAT)

    def in_copy(c, slot):
        return pltpu.make_async_copy(
            aflat.at[pl.ds(c * _ROWS, _ROWS)], ibuf.at[slot],
            in_sem.at[slot])

    def out_copy(c, slot):
        return pltpu.make_async_copy(
            obuf.at[slot], oflat.at[pl.ds(c * _ROWS, _ROWS)],
            out_sem.at[slot])

    for k in range(_K):
        in_copy(k, k).start()

    def body(j, carry):
        slot = lax.rem(j, _K)
        in_copy(j, slot).wait()

        @pl.when(j >= _K)
        def _():
            out_copy(j - _K, slot).wait()

        obuf[slot] = ibuf[slot] + bias_v[...]
        out_copy(j, slot).start()

        @pl.when(j + _K < _NCH)
        def _():
            in_copy(j + _K, slot).start()

        return carry

    lax.fori_loop(0, _NCH, body, 0)
    for k in range(_K):
        c = _NCH - _K + k
        out_copy(c, c % _K).wait()


def _add_body_t(attn_ref, bias_ref, out_ref):
    out_ref[...] = attn_ref[...] + bias_ref[...][..., None]


def _tc_add_t(attn_t, bias3, b, ich=72):
    return pl.pallas_call(
        _add_body_t,
        grid=(_NH, 144 // ich),
        in_specs=[
            pl.BlockSpec((1, ich, 144, b), lambda h, i: (h, i, 0, 0)),
            pl.BlockSpec((1, ich, 144), lambda h, i: (h, i, 0)),
        ],
        out_specs=pl.BlockSpec((1, ich, 144, b), lambda h, i: (h, i, 0, 0)),
        out_shape=jax.ShapeDtypeStruct((_NH, 144, 144, b), jnp.float32),
        compiler_params=pltpu.CompilerParams(
            dimension_semantics=("arbitrary", "arbitrary")),
    )(attn_t, bias3)


def kernel(attn, earth_specific_bias, position_index, window_type_index):
    w = jnp.asarray(window_type_index, jnp.int32)
    esb_t = jnp.transpose(earth_specific_bias, (2, 0, 1))
    table = lax.dynamic_index_in_dim(esb_t, w, axis=2, keepdims=False)
    bias_flat = _sc_gather(table.reshape(-1), position_index.astype(jnp.int32))
    b = attn.shape[0]
    attn_t = jnp.transpose(attn, (1, 2, 3, 0))
    out_t = _tc_add_t(attn_t, bias_flat.reshape(_NH, 144, 144), b)
    return jnp.transpose(out_t, (3, 0, 1, 2))


# speedup vs baseline: 13.3914x; 1.0115x over previous
import jax
import jax.numpy as jnp
from jax import lax
from jax.experimental import pallas as pl
from jax.experimental.pallas import tpu as pltpu
from jax.experimental.pallas import tpu_sc as plsc

_NH = 12
_NN = 20736
_NU = 3312
_FLAT = _NH * _NN
_NC = 2
_CHUNK = _NN // 2
_NWORK = 2 * _NH


def _sc_gather_body(table_hbm, idx_hbm, out_hbm, table_v, idx_v, out_v):
    wid = lax.axis_index("s") * _NC + lax.axis_index("c")

    @pl.when(wid < _NWORK)
    def _():
        h = wid // 2
        c0 = (wid % 2) * _CHUNK
        pltpu.sync_copy(table_hbm.at[pl.ds(h * _NU, _NU)], table_v)
        pltpu.sync_copy(idx_hbm.at[pl.ds(c0, _CHUNK)], idx_v)

        def body(j, carry):
            ivec = idx_v[pl.ds(j * 16, 16)]
            out_v[pl.ds(j * 16, 16)] = plsc.load_gather(table_v, [ivec])
            return carry

        lax.fori_loop(0, _CHUNK // 16, body, 0)
        pltpu.sync_copy(out_v, out_hbm.at[pl.ds(h * _NN + c0, _CHUNK)])


_sc_gather = pl.kernel(
    _sc_gather_body,
    out_type=jax.ShapeDtypeStruct((_FLAT,), jnp.float32),
    mesh=plsc.VectorSubcoreMesh(core_axis_name="c", subcore_axis_name="s"),
    compiler_params=pltpu.CompilerParams(needs_layout_passes=False),
    scratch_types=[
        pltpu.VMEM((_NU,), jnp.float32),
        pltpu.VMEM((_CHUNK,), jnp.int32),
        pltpu.VMEM((_CHUNK,), jnp.float32),
    ],
)


_ROWS = 2
_NCH = 240 // _ROWS
_K = 8


def _add_manual_body(attn_hbm, bias_v, out_hbm, ibuf, obuf, in_sem, out_sem):
    aflat = attn_hbm.reshape(240, _FLAT)
    oflat = out_hbm.reshape(240, _FLAT)

    def in_copy(c, slot):
        return pltpu.make_async_copy(
            aflat.at[pl.ds(c * _ROWS, _ROWS)], ibuf.at[slot],
            in_sem.at[slot])

    def out_copy(c, slot):
        return pltpu.make_async_copy(
            obuf.at[slot], oflat.at[pl.ds(c * _ROWS, _ROWS)],
            out_sem.at[slot])

    for k in range(_K):
        in_copy(k, k).start()

    def body(j, carry):
        slot = lax.rem(j, _K)
        in_copy(j, slot).wait()

        @pl.when(j >= _K)
        def _():
            out_copy(j - _K, slot).wait()

        obuf[slot] = ibuf[slot] + bias_v[...]
        out_copy(j, slot).start()

        @pl.when(j + _K < _NCH)
        def _():
            in_copy(j + _K, slot).start()

        return carry

    lax.fori_loop(0, _NCH, body, 0)
    for k in range(_K):
        c = _NCH - _K + k
        out_copy(c, c % _K).wait()


def _add_body_t(attn_ref, bias_ref, out_ref):
    out_ref[...] = attn_ref[...] + bias_ref[...][..., None]


def _tc_add_t(attn_t, bias3, b, ich=72):
    return pl.pallas_call(
        _add_body_t,
        grid=(_NH, 144 // ich),
        in_specs=[
            pl.BlockSpec((1, ich, 144, b), lambda h, i: (h, i, 0, 0)),
            pl.BlockSpec((1, ich, 144), lambda h, i: (h, i, 0)),
        ],
        out_specs=pl.BlockSpec((1, ich, 144, b), lambda h, i: (h, i, 0, 0)),
        out_shape=jax.ShapeDtypeStruct((_NH, 144, 144, b), jnp.float32),
        compiler_params=pltpu.CompilerParams(
            dimension_semantics=("arbitrary", "arbitrary")),
    )(attn_t, bias3)


def kernel(attn, earth_specific_bias, position_index, window_type_index):
    w = jnp.asarray(window_type_index, jnp.int32)
    esb_t = jnp.transpose(earth_specific_bias, (2, 0, 1))
    table = lax.dynamic_index_in_dim(esb_t, w, axis=2, keepdims=False)
    bias_flat = _sc_gather(table.reshape(-1), position_index.astype(jnp.int32))
    b = attn.shape[0]
    attn_t = jnp.transpose(attn, (1, 2, 3, 0))
    out_t = _tc_add_t(attn_t, bias_flat.reshape(_NH, 144, 144), b)
    return jnp.transpose(out_t, (3, 0, 1, 2))
